# baseline (speedup 1.0000x reference)
import jax
import jax.numpy as jnp
from jax.experimental import pallas as pl
from jax.experimental.pallas import tpu as pltpu

N = 4096
D = 64
BM = 512


def _matmul_block(inp_ref, w_ref, out_ref):
    i = pl.program_id(0)
    out_ref[pl.ds(i * BM, BM), :] = jnp.dot(
        inp_ref[...], w_ref[...], preferred_element_type=jnp.float32)


@jax.jit
def kernel(inp, weight):
    grid = (N // BM,)
    return pl.pallas_call(
        _matmul_block,
        grid=grid,
        in_specs=[
            pl.BlockSpec((BM, N), lambda i: (i, 0)),
            pl.BlockSpec((N, D), lambda i: (0, 0)),
        ],
        out_specs=pl.BlockSpec((N, D), lambda i: (0, 0)),
        out_shape=jax.ShapeDtypeStruct((N, D), jnp.float32),
        compiler_params=pltpu.CompilerParams(
            skip_device_barrier=True,
            disable_bounds_checks=True,
        ),
    )(inp, weight)


# speedup vs baseline: 1.0077x; 1.0077x over previous
import jax
import jax.numpy as jnp
from jax.experimental import pallas as pl
from jax.experimental.pallas import tpu as pltpu

N = 4096
D = 64
BM = 512


def _matmul_block(inp_ref, w_ref, out_ref):
    i = pl.program_id(0)
    out_ref[pl.ds(i * BM, BM), :] = jnp.dot(
        inp_ref[0], w_ref[...], preferred_element_type=jnp.float32)


@jax.jit
def kernel(inp, weight):
    inp3 = inp.reshape(N // BM, BM, N)
    grid = (N // BM,)
    return pl.pallas_call(
        _matmul_block,
        grid=grid,
        in_specs=[
            pl.BlockSpec((1, BM, N), lambda i: (i, 0, 0)),
            pl.BlockSpec((N, D), lambda i: (0, 0)),
        ],
        out_specs=pl.BlockSpec((N, D), lambda i: (0, 0)),
        out_shape=jax.ShapeDtypeStruct((N, D), jnp.float32),
        compiler_params=pltpu.CompilerParams(
            skip_device_barrier=True,
            disable_bounds_checks=True,
        ),
    )(inp3, weight)


# speedup vs baseline: 1.0132x; 1.0054x over previous
import jax
import jax.numpy as jnp
from jax.experimental import pallas as pl
from jax.experimental.pallas import tpu as pltpu

N = 4096
D = 64
BM = 512


def _matmul_block(inp_ref, w_ref, out_ref):
    i = pl.program_id(0)
    out_ref[pl.ds(i * BM, BM), :] = jnp.dot(
        inp_ref[...], w_ref[...], preferred_element_type=jnp.float32)


@jax.jit
def kernel(inp, weight):
    grid = (N // BM,)
    return pl.pallas_call(
        _matmul_block,
        grid=grid,
        in_specs=[
            pl.BlockSpec((BM, N), lambda i: (i, 0)),
            pl.BlockSpec((N, D), lambda i: (0, 0)),
        ],
        out_specs=pl.BlockSpec((N, D), lambda i: (0, 0)),
        out_shape=jax.ShapeDtypeStruct((N, D), jnp.float32),
        compiler_params=pltpu.CompilerParams(
            skip_device_barrier=True,
            disable_bounds_checks=True,
        ),
    )(inp, weight)
